# baseline (speedup 1.0000x reference)
import functools

import jax
import jax.numpy as jnp
from jax import lax
from jax.experimental import pallas as pl
from jax.experimental.pallas import tpu as pltpu
from jax.experimental.pallas import tpu_sc as plsc

NC = 2
NS = 16
K = 128
PAD = 8


@functools.lru_cache(maxsize=None)
def _make_sc_agg(N, D, R, EP):
    assert R % NC == 0
    rel_per_core = R // NC
    e_per_tile = EP // NS
    assert e_per_tile % K == 0
    n_chunks = e_per_tile // K
    assert n_chunks % 2 == 0
    rpt = (N // NS) // 8 * 8
    tail = N - NS * rpt
    assert tail % 8 == 0 and tail // 8 <= NS
    n_z128 = rpt // K
    z_rem = rpt - n_z128 * K

    mesh = plsc.VectorSubcoreMesh(core_axis_name="c", subcore_axis_name="s")

    scratch = [
        pltpu.VMEM((2, K), jnp.int32),
        pltpu.VMEM((2, K), jnp.int32),
        pltpu.VMEM((K, D), jnp.float32),
        pltpu.VMEM_SHARED((N, D), jnp.float32),
        pltpu.SemaphoreType.DMA,
        pltpu.SemaphoreType.DMA,
    ]

    @functools.partial(
        pl.kernel,
        mesh=mesh,
        out_type=jax.ShapeDtypeStruct((R, N, D), jnp.float32),
        scratch_types=scratch,
    )
    def sc_agg(x_hbm, ei_hbm, z_hbm, out_hbm, ib0, ib1, rows_v, acc_sh,
               sem_g, sem_i):
        c = lax.axis_index("c")
        s = lax.axis_index("s")
        row0 = s * rpt
        trow = NS * rpt + s * 8

        for phase in range(rel_per_core):
            r = c * rel_per_core + phase

            for z in range(n_z128):
                pltpu.sync_copy(z_hbm, acc_sh.at[pl.ds(row0 + z * K, K)])
            if z_rem:
                pltpu.sync_copy(z_hbm.at[pl.ds(0, z_rem)],
                                acc_sh.at[pl.ds(row0 + n_z128 * K, z_rem)])
            if tail:
                @pl.when(s < tail // 8)
                def _():
                    pltpu.sync_copy(z_hbm.at[pl.ds(0, 8)],
                                    acc_sh.at[pl.ds(trow, 8)])
            plsc.subcore_barrier()

            base = (r * NS + s) * n_chunks
            pltpu.sync_copy(ei_hbm.at[base], ib0)

            def _pair(g, carry):
                for b, (cur, nxt) in enumerate(((ib0, ib1), (ib1, ib0))):
                    j = 2 * g + b
                    pltpu.async_copy(ei_hbm.at[base + j + 1], nxt, sem_i)
                    pltpu.async_copy(x_hbm.at[cur.at[0]], rows_v,
                                     sem_g).wait()
                    pltpu.sync_copy(rows_v, acc_sh.at[cur.at[1]], add=True)
                    pltpu.make_async_copy(ei_hbm.at[base], nxt, sem_i).wait()
                return carry
            lax.fori_loop(0, n_chunks // 2, _pair, 0)
            plsc.subcore_barrier()

            pltpu.sync_copy(acc_sh.at[pl.ds(row0, rpt)],
                            out_hbm.at[r, pl.ds(row0, rpt)])
            if tail:
                @pl.when(s < tail // 8)
                def _():
                    pltpu.sync_copy(acc_sh.at[pl.ds(trow, 8)],
                                    out_hbm.at[r, pl.ds(trow, 8)])

    return sc_agg


@functools.lru_cache(maxsize=None)
def _make_tc_mm(N, D, Dout, R, bm=1000):
    grid = N // bm

    def _mm_body(x_ref, a_ref, w_ref, ws_ref, o_ref):
        acc = jnp.dot(x_ref[...], ws_ref[...],
                      preferred_element_type=jnp.float32)
        for r in range(R):
            acc = acc + jnp.dot(a_ref[r], w_ref[r],
                                preferred_element_type=jnp.float32)
        o_ref[...] = acc

    return pl.pallas_call(
        _mm_body,
        grid=(grid,),
        in_specs=[
            pl.BlockSpec((bm, D), lambda i: (i, 0)),
            pl.BlockSpec((R, bm, D), lambda i: (0, i, 0)),
            pl.BlockSpec((R, D, Dout), lambda i: (0, 0, 0)),
            pl.BlockSpec((D, Dout), lambda i: (0, 0)),
        ],
        out_specs=pl.BlockSpec((bm, Dout), lambda i: (i, 0)),
        out_shape=jax.ShapeDtypeStruct((N, Dout), jnp.float32),
    )


def kernel(x, edge_index, W, W_self):
    B, N, D = x.shape
    R, _, E = edge_index.shape
    Dout = W_self.shape[1]
    x2 = x.reshape(N, D)

    ept = -(-E // (NS * K)) * K
    EP = ept * NS
    n_chunks = ept // K
    npad = ept - E // NS
    ei = edge_index.reshape(R, 2, NS, E // NS)
    pad_src = jnp.full((R, 1, NS, npad), N, jnp.int32)
    pad_dst = jnp.broadcast_to(
        (jnp.arange(npad, dtype=jnp.int32) * 64 % N).reshape(1, 1, 1, npad),
        (R, 1, NS, npad))
    ei = jnp.concatenate(
        [ei, jnp.concatenate([pad_src, pad_dst], axis=1)], axis=-1)
    ei = ei.reshape(R, 2, NS, n_chunks, K).transpose(0, 2, 3, 1, 4)
    ei = ei.reshape(R * NS * n_chunks, 2, K)
    ei = jnp.concatenate([ei, jnp.zeros((1, 2, K), jnp.int32)])

    xp = jnp.concatenate([x2, jnp.zeros((PAD, D), jnp.float32)])
    zeros = jnp.zeros((K, D), jnp.float32)

    agg = _make_sc_agg(N, D, R, EP)(xp, ei, zeros)
    out = _make_tc_mm(N, D, Dout, R)(x2, agg, W, W_self)
    return out.reshape(B, N, Dout)


# speedup vs baseline: 1.8074x; 1.8074x over previous
import functools

import jax
import jax.numpy as jnp
from jax import lax
from jax.experimental import pallas as pl
from jax.experimental.pallas import tpu as pltpu
from jax.experimental.pallas import tpu_sc as plsc

NC = 2
NS = 16
K = 128


@functools.lru_cache(maxsize=None)
def _make_sc_agg(N, D, R, E):
    assert R % NC == 0
    rel_per_core = R // NC
    e_per_tile = E // NS
    n_full = e_per_tile // K
    k_rem = e_per_tile - n_full * K
    rpt = (N // NS) // 8 * 8
    tail = N - NS * rpt
    assert tail % 8 == 0 and tail // 8 <= NS
    n_z128 = rpt // K
    z_rem = rpt - n_z128 * K

    mesh = plsc.VectorSubcoreMesh(core_axis_name="c", subcore_axis_name="s")

    scratch = [
        pltpu.VMEM((K,), jnp.int32),
        pltpu.VMEM((K,), jnp.int32),
        pltpu.VMEM((K, D), jnp.float32),
        pltpu.VMEM_SHARED((N, D), jnp.float32),
        pltpu.SemaphoreType.DMA,
    ]
    if k_rem:
        scratch += [
            pltpu.VMEM((k_rem,), jnp.int32),
            pltpu.VMEM((k_rem,), jnp.int32),
            pltpu.VMEM((k_rem, D), jnp.float32),
        ]

    @functools.partial(
        pl.kernel,
        mesh=mesh,
        out_type=jax.ShapeDtypeStruct((R, N, D), jnp.float32),
        scratch_types=scratch,
    )
    def sc_agg(x_hbm, ei_hbm, out_hbm, src_v, dst_v, rows_v, acc_sh, sem,
               *rem_bufs):
        c = lax.axis_index("c")
        s = lax.axis_index("s")
        ebase = s * e_per_tile
        row0 = s * rpt
        trow = NS * rpt + s * 8

        for phase in range(rel_per_core):
            r = c * rel_per_core + phase

            def _zrow(i, carry):
                for j in range(D // 16):
                    rows_v[i, pl.ds(j * 16, 16)] = jnp.zeros((16,), jnp.float32)
                return carry
            lax.fori_loop(0, K, _zrow, 0)
            for z in range(n_z128):
                pltpu.sync_copy(
                    rows_v,
                    acc_sh.at[pl.ds(row0 + z * K, K)])
            if z_rem:
                pltpu.sync_copy(
                    rows_v.at[pl.ds(0, z_rem)],
                    acc_sh.at[pl.ds(row0 + n_z128 * K, z_rem)])
            if tail:
                @pl.when(s < tail // 8)
                def _():
                    pltpu.sync_copy(rows_v.at[pl.ds(0, 8)],
                                    acc_sh.at[pl.ds(trow, 8)])
            plsc.subcore_barrier()

            src0 = (2 * r) * E + ebase
            dst0 = (2 * r + 1) * E + ebase

            def _chunk(j, carry):
                off = j * K
                pltpu.sync_copy(ei_hbm.at[pl.ds(src0 + off, K)], src_v)
                pltpu.sync_copy(ei_hbm.at[pl.ds(dst0 + off, K)], dst_v)
                pltpu.async_copy(x_hbm.at[src_v], rows_v, sem).wait()
                pltpu.sync_copy(rows_v, acc_sh.at[dst_v], add=True)
                return carry
            lax.fori_loop(0, n_full, _chunk, 0)
            if k_rem:
                srcr_v, dstr_v, rowsr_v = rem_bufs
                off = n_full * K
                pltpu.sync_copy(ei_hbm.at[pl.ds(src0 + off, k_rem)], srcr_v)
                pltpu.sync_copy(ei_hbm.at[pl.ds(dst0 + off, k_rem)], dstr_v)
                pltpu.async_copy(x_hbm.at[srcr_v], rowsr_v, sem).wait()
                pltpu.sync_copy(rowsr_v, acc_sh.at[dstr_v], add=True)
            plsc.subcore_barrier()

            pltpu.sync_copy(
                acc_sh.at[pl.ds(row0, rpt)],
                out_hbm.at[r, pl.ds(row0, rpt)])
            if tail:
                @pl.when(s < tail // 8)
                def _():
                    pltpu.sync_copy(acc_sh.at[pl.ds(trow, 8)],
                                    out_hbm.at[r, pl.ds(trow, 8)])

    return sc_agg


@functools.lru_cache(maxsize=None)
def _make_tc_mm(N, D, Dout, R, bm=1000):
    grid = N // bm

    def _mm_body(x_ref, a_ref, w_ref, ws_ref, o_ref):
        acc = jnp.dot(x_ref[...], ws_ref[...],
                      preferred_element_type=jnp.float32)
        for r in range(R):
            acc = acc + jnp.dot(a_ref[r], w_ref[r],
                                preferred_element_type=jnp.float32)
        o_ref[...] = acc

    return pl.pallas_call(
        _mm_body,
        grid=(grid,),
        in_specs=[
            pl.BlockSpec((bm, D), lambda i: (i, 0)),
            pl.BlockSpec((R, bm, D), lambda i: (0, i, 0)),
            pl.BlockSpec((R, D, Dout), lambda i: (0, 0, 0)),
            pl.BlockSpec((D, Dout), lambda i: (0, 0)),
        ],
        out_specs=pl.BlockSpec((bm, Dout), lambda i: (i, 0)),
        out_shape=jax.ShapeDtypeStruct((N, Dout), jnp.float32),
    )


def kernel(x, edge_index, W, W_self):
    B, N, D = x.shape
    R, _, E = edge_index.shape
    Dout = W_self.shape[1]
    x2 = x.reshape(N, D)
    agg = _make_sc_agg(N, D, R, E)(x2, edge_index.reshape(-1))
    out = _make_tc_mm(N, D, Dout, R)(x2, agg, W, W_self)
    return out.reshape(B, N, Dout)


# speedup vs baseline: 1.9643x; 1.0868x over previous
import functools

import jax
import jax.numpy as jnp
from jax import lax
from jax.experimental import pallas as pl
from jax.experimental.pallas import tpu as pltpu
from jax.experimental.pallas import tpu_sc as plsc

NC = 2
NS = 16
K = 128


@functools.lru_cache(maxsize=None)
def _make_sc_agg(N, D, R, E):
    assert R % NC == 0
    rel_per_core = R // NC
    e_per_tile = E // NS
    n_full = e_per_tile // K
    k_rem = e_per_tile - n_full * K
    rpt = (N // NS) // 8 * 8
    tail = N - NS * rpt
    assert tail % 8 == 0 and tail // 8 <= NS
    n_z128 = rpt // K
    z_rem = rpt - n_z128 * K

    mesh = plsc.VectorSubcoreMesh(core_axis_name="c", subcore_axis_name="s")

    G = 3
    n_groups = n_full // G
    g_rem = n_full - n_groups * G

    scratch = [
        pltpu.VMEM((G * K,), jnp.int32),
        pltpu.VMEM((K,), jnp.int32),
        pltpu.VMEM((K, D), jnp.float32),
        pltpu.VMEM_SHARED((N, D), jnp.float32),
        pltpu.SemaphoreType.DMA,
    ]
    if k_rem:
        scratch += [
            pltpu.VMEM((k_rem,), jnp.int32),
            pltpu.VMEM((k_rem,), jnp.int32),
            pltpu.VMEM((k_rem, D), jnp.float32),
        ]

    @functools.partial(
        pl.kernel,
        mesh=mesh,
        out_type=jax.ShapeDtypeStruct((R, N, D), jnp.float32),
        scratch_types=scratch,
    )
    def sc_agg(x_hbm, ei_hbm, out_hbm, src_v, dst_v, rows_v, acc_sh, sem,
               *rem_bufs):
        c = lax.axis_index("c")
        s = lax.axis_index("s")
        ebase = s * e_per_tile
        row0 = s * rpt
        trow = NS * rpt + s * 8

        for phase in range(rel_per_core):
            r = c * rel_per_core + phase

            def _zrow(i, carry):
                for j in range(D // 16):
                    rows_v[i, pl.ds(j * 16, 16)] = jnp.zeros((16,), jnp.float32)
                return carry
            lax.fori_loop(0, K, _zrow, 0)
            for z in range(n_z128):
                pltpu.sync_copy(
                    rows_v,
                    acc_sh.at[pl.ds(row0 + z * K, K)])
            if z_rem:
                pltpu.sync_copy(
                    rows_v.at[pl.ds(0, z_rem)],
                    acc_sh.at[pl.ds(row0 + n_z128 * K, z_rem)])
            if tail:
                @pl.when(s < tail // 8)
                def _():
                    pltpu.sync_copy(rows_v.at[pl.ds(0, 8)],
                                    acc_sh.at[pl.ds(trow, 8)])
            plsc.subcore_barrier()

            src0 = (2 * r) * E + ebase
            dst0 = (2 * r + 1) * E + ebase

            def _group(g, carry):
                goff = g * (G * K)
                pltpu.sync_copy(ei_hbm.at[pl.ds(src0 + goff, G * K)], src_v)
                for b in range(G):
                    off = goff + b * K
                    pltpu.sync_copy(ei_hbm.at[pl.ds(dst0 + off, K)], dst_v)
                    pltpu.async_copy(x_hbm.at[src_v.at[pl.ds(b * K, K)]],
                                     rows_v, sem).wait()
                    pltpu.sync_copy(rows_v, acc_sh.at[dst_v], add=True)
                return carry
            lax.fori_loop(0, n_groups, _group, 0)
            for b in range(g_rem):
                off = (n_groups * G + b) * K
                pltpu.sync_copy(ei_hbm.at[pl.ds(src0 + off, K)],
                                src_v.at[pl.ds(0, K)])
                pltpu.sync_copy(ei_hbm.at[pl.ds(dst0 + off, K)], dst_v)
                pltpu.async_copy(x_hbm.at[src_v.at[pl.ds(0, K)]], rows_v,
                                 sem).wait()
                pltpu.sync_copy(rows_v, acc_sh.at[dst_v], add=True)
            if k_rem:
                srcr_v, dstr_v, rowsr_v = rem_bufs
                off = n_full * K
                pltpu.sync_copy(ei_hbm.at[pl.ds(src0 + off, k_rem)], srcr_v)
                pltpu.sync_copy(ei_hbm.at[pl.ds(dst0 + off, k_rem)], dstr_v)
                pltpu.async_copy(x_hbm.at[srcr_v], rowsr_v, sem).wait()
                pltpu.sync_copy(rowsr_v, acc_sh.at[dstr_v], add=True)
            plsc.subcore_barrier()

            pltpu.sync_copy(
                acc_sh.at[pl.ds(row0, rpt)],
                out_hbm.at[r, pl.ds(row0, rpt)])
            if tail:
                @pl.when(s < tail // 8)
                def _():
                    pltpu.sync_copy(acc_sh.at[pl.ds(trow, 8)],
                                    out_hbm.at[r, pl.ds(trow, 8)])

    return sc_agg


@functools.lru_cache(maxsize=None)
def _make_tc_self(N, D, Dout, bm=1000):
    def _body(x_ref, ws_ref, o_ref):
        o_ref[...] = jnp.dot(x_ref[...], ws_ref[...],
                             preferred_element_type=jnp.float32)

    return pl.pallas_call(
        _body,
        grid=(N // bm,),
        in_specs=[
            pl.BlockSpec((bm, D), lambda i: (i, 0)),
            pl.BlockSpec((D, Dout), lambda i: (0, 0)),
        ],
        out_specs=pl.BlockSpec((bm, Dout), lambda i: (i, 0)),
        out_shape=jax.ShapeDtypeStruct((N, Dout), jnp.float32),
    )


@functools.lru_cache(maxsize=None)
def _make_tc_sum(N, D, Dout, R, bm=1000):
    def _body(s_ref, a_ref, w_ref, o_ref):
        acc = s_ref[...]
        for r in range(R):
            acc = acc + jnp.dot(a_ref[r], w_ref[r],
                                preferred_element_type=jnp.float32)
        o_ref[...] = acc

    return pl.pallas_call(
        _body,
        grid=(N // bm,),
        in_specs=[
            pl.BlockSpec((bm, Dout), lambda i: (i, 0)),
            pl.BlockSpec((R, bm, D), lambda i: (0, i, 0)),
            pl.BlockSpec((R, D, Dout), lambda i: (0, 0, 0)),
        ],
        out_specs=pl.BlockSpec((bm, Dout), lambda i: (i, 0)),
        out_shape=jax.ShapeDtypeStruct((N, Dout), jnp.float32),
    )


def kernel(x, edge_index, W, W_self):
    B, N, D = x.shape
    R, _, E = edge_index.shape
    Dout = W_self.shape[1]
    x2 = x.reshape(N, D)
    self_o = _make_tc_self(N, D, Dout)(x2, W_self)
    agg = _make_sc_agg(N, D, R, E)(x2, edge_index.reshape(-1))
    out = _make_tc_sum(N, D, Dout, R)(self_o, agg, W)
    return out.reshape(B, N, Dout)


# speedup vs baseline: 2.0334x; 1.0352x over previous
import functools

import jax
import jax.numpy as jnp
from jax import lax
from jax.experimental import pallas as pl
from jax.experimental.pallas import tpu as pltpu
from jax.experimental.pallas import tpu_sc as plsc

NC = 2
NS = 16
K = 128


@functools.lru_cache(maxsize=None)
def _make_sc_agg(N, D, R, E):
    assert R % NC == 0
    rel_per_core = R // NC
    e_per_tile = E // NS
    n_full = e_per_tile // K
    k_rem = e_per_tile - n_full * K
    rpt = (N // NS) // 8 * 8
    tail = N - NS * rpt
    assert tail % 8 == 0 and tail // 8 <= NS
    n_z128 = rpt // K
    z_rem = rpt - n_z128 * K

    mesh = plsc.VectorSubcoreMesh(core_axis_name="c", subcore_axis_name="s")

    G = 13
    n_groups = n_full // G
    g_rem = n_full - n_groups * G

    scratch = [
        pltpu.VMEM((G * K,), jnp.int32),
        pltpu.VMEM((K,), jnp.int32),
        pltpu.VMEM((K, D), jnp.float32),
        pltpu.VMEM_SHARED((N, D), jnp.float32),
        pltpu.SemaphoreType.DMA,
    ]
    if k_rem:
        scratch += [
            pltpu.VMEM((k_rem,), jnp.int32),
            pltpu.VMEM((k_rem,), jnp.int32),
            pltpu.VMEM((k_rem, D), jnp.float32),
        ]

    @functools.partial(
        pl.kernel,
        mesh=mesh,
        out_type=jax.ShapeDtypeStruct((R, N, D), jnp.float32),
        scratch_types=scratch,
    )
    def sc_agg(x_hbm, ei_hbm, out_hbm, src_v, dst_v, rows_v, acc_sh, sem,
               *rem_bufs):
        c = lax.axis_index("c")
        s = lax.axis_index("s")
        ebase = s * e_per_tile
        row0 = s * rpt
        trow = NS * rpt + s * 8

        for phase in range(rel_per_core):
            r = c * rel_per_core + phase

            def _zrow(i, carry):
                for j in range(D // 16):
                    rows_v[i, pl.ds(j * 16, 16)] = jnp.zeros((16,), jnp.float32)
                return carry
            lax.fori_loop(0, K, _zrow, 0)
            for z in range(n_z128):
                pltpu.sync_copy(
                    rows_v,
                    acc_sh.at[pl.ds(row0 + z * K, K)])
            if z_rem:
                pltpu.sync_copy(
                    rows_v.at[pl.ds(0, z_rem)],
                    acc_sh.at[pl.ds(row0 + n_z128 * K, z_rem)])
            if tail:
                @pl.when(s < tail // 8)
                def _():
                    pltpu.sync_copy(rows_v.at[pl.ds(0, 8)],
                                    acc_sh.at[pl.ds(trow, 8)])
            plsc.subcore_barrier()

            src0 = (2 * r) * E + ebase
            dst0 = (2 * r + 1) * E + ebase

            def _group(g, carry):
                goff = g * (G * K)
                pltpu.sync_copy(ei_hbm.at[pl.ds(src0 + goff, G * K)], src_v)
                for b in range(G):
                    off = goff + b * K
                    pltpu.sync_copy(ei_hbm.at[pl.ds(dst0 + off, K)], dst_v)
                    pltpu.async_copy(x_hbm.at[src_v.at[pl.ds(b * K, K)]],
                                     rows_v, sem).wait()
                    pltpu.sync_copy(rows_v, acc_sh.at[dst_v], add=True)
                return carry
            lax.fori_loop(0, n_groups, _group, 0)
            for b in range(g_rem):
                off = (n_groups * G + b) * K
                pltpu.sync_copy(ei_hbm.at[pl.ds(src0 + off, K)],
                                src_v.at[pl.ds(0, K)])
                pltpu.sync_copy(ei_hbm.at[pl.ds(dst0 + off, K)], dst_v)
                pltpu.async_copy(x_hbm.at[src_v.at[pl.ds(0, K)]], rows_v,
                                 sem).wait()
                pltpu.sync_copy(rows_v, acc_sh.at[dst_v], add=True)
            if k_rem:
                srcr_v, dstr_v, rowsr_v = rem_bufs
                off = n_full * K
                pltpu.sync_copy(ei_hbm.at[pl.ds(src0 + off, k_rem)], srcr_v)
                pltpu.sync_copy(ei_hbm.at[pl.ds(dst0 + off, k_rem)], dstr_v)
                pltpu.async_copy(x_hbm.at[srcr_v], rowsr_v, sem).wait()
                pltpu.sync_copy(rowsr_v, acc_sh.at[dstr_v], add=True)
            plsc.subcore_barrier()

            pltpu.sync_copy(
                acc_sh.at[pl.ds(row0, rpt)],
                out_hbm.at[r, pl.ds(row0, rpt)])
            if tail:
                @pl.when(s < tail // 8)
                def _():
                    pltpu.sync_copy(acc_sh.at[pl.ds(trow, 8)],
                                    out_hbm.at[r, pl.ds(trow, 8)])

    return sc_agg


@functools.lru_cache(maxsize=None)
def _make_tc_self(N, D, Dout, bm=1000):
    def _body(x_ref, ws_ref, o_ref):
        o_ref[...] = jnp.dot(x_ref[...], ws_ref[...],
                             preferred_element_type=jnp.float32)

    return pl.pallas_call(
        _body,
        grid=(N // bm,),
        in_specs=[
            pl.BlockSpec((bm, D), lambda i: (i, 0)),
            pl.BlockSpec((D, Dout), lambda i: (0, 0)),
        ],
        out_specs=pl.BlockSpec((bm, Dout), lambda i: (i, 0)),
        out_shape=jax.ShapeDtypeStruct((N, Dout), jnp.float32),
    )


@functools.lru_cache(maxsize=None)
def _make_tc_sum(N, D, Dout, R, bm=1000):
    def _body(s_ref, a_ref, w_ref, o_ref):
        acc = s_ref[...]
        for r in range(R):
            acc = acc + jnp.dot(a_ref[r], w_ref[r],
                                preferred_element_type=jnp.float32)
        o_ref[...] = acc

    return pl.pallas_call(
        _body,
        grid=(N // bm,),
        in_specs=[
            pl.BlockSpec((bm, Dout), lambda i: (i, 0)),
            pl.BlockSpec((R, bm, D), lambda i: (0, i, 0)),
            pl.BlockSpec((R, D, Dout), lambda i: (0, 0, 0)),
        ],
        out_specs=pl.BlockSpec((bm, Dout), lambda i: (i, 0)),
        out_shape=jax.ShapeDtypeStruct((N, Dout), jnp.float32),
    )


def kernel(x, edge_index, W, W_self):
    B, N, D = x.shape
    R, _, E = edge_index.shape
    Dout = W_self.shape[1]
    x2 = x.reshape(N, D)
    self_o = _make_tc_self(N, D, Dout)(x2, W_self)
    agg = _make_sc_agg(N, D, R, E)(x2, edge_index.reshape(-1))
    out = _make_tc_sum(N, D, Dout, R)(self_o, agg, W)
    return out.reshape(B, N, Dout)


# speedup vs baseline: 2.3660x; 1.1636x over previous
import functools

import jax
import jax.numpy as jnp
from jax import lax
from jax.experimental import pallas as pl
from jax.experimental.pallas import tpu as pltpu
from jax.experimental.pallas import tpu_sc as plsc

NC = 2
NS = 16
K = 128


@functools.lru_cache(maxsize=None)
def _make_sc_agg(N, D, R, E):
    assert R % NC == 0
    rel_per_core = R // NC
    e_per_tile = E // NS
    n_full = e_per_tile // K
    k_rem = e_per_tile - n_full * K
    rpt = (N // NS) // 8 * 8
    tail = N - NS * rpt
    assert tail % 8 == 0 and tail // 8 <= NS
    n_z128 = rpt // K
    z_rem = rpt - n_z128 * K

    mesh = plsc.VectorSubcoreMesh(core_axis_name="c", subcore_axis_name="s")

    scratch = [
        pltpu.VMEM((n_full * K,), jnp.int32),
        pltpu.VMEM((K,), jnp.int32),
        pltpu.VMEM((K,), jnp.int32),
        pltpu.VMEM((K, D), jnp.float32),
        pltpu.VMEM_SHARED((N, D), jnp.float32),
        pltpu.SemaphoreType.DMA,
        pltpu.SemaphoreType.DMA,
    ]
    if k_rem:
        scratch += [
            pltpu.VMEM((k_rem,), jnp.int32),
            pltpu.VMEM((k_rem,), jnp.int32),
            pltpu.VMEM((k_rem, D), jnp.float32),
        ]

    @functools.partial(
        pl.kernel,
        mesh=mesh,
        out_type=jax.ShapeDtypeStruct((R, N, D), jnp.float32),
        scratch_types=scratch,
    )
    def sc_agg(x_hbm, ei_hbm, out_hbm, src_v, dst_a, dst_b, rows_v, acc_sh,
               sem, sem_i, *rem_bufs):
        dstb = (dst_a, dst_b)
        c = lax.axis_index("c")
        s = lax.axis_index("s")
        ebase = s * e_per_tile
        row0 = s * rpt
        trow = NS * rpt + s * 8

        for phase in range(rel_per_core):
            r = c * rel_per_core + phase

            def _zrow(i, carry):
                for j in range(D // 16):
                    rows_v[i, pl.ds(j * 16, 16)] = jnp.zeros((16,), jnp.float32)
                return carry
            lax.fori_loop(0, K, _zrow, 0)
            for z in range(n_z128):
                pltpu.sync_copy(
                    rows_v,
                    acc_sh.at[pl.ds(row0 + z * K, K)])
            if z_rem:
                pltpu.sync_copy(
                    rows_v.at[pl.ds(0, z_rem)],
                    acc_sh.at[pl.ds(row0 + n_z128 * K, z_rem)])
            if tail:
                @pl.when(s < tail // 8)
                def _():
                    pltpu.sync_copy(rows_v.at[pl.ds(0, 8)],
                                    acc_sh.at[pl.ds(trow, 8)])
            plsc.subcore_barrier()

            src0 = (2 * r) * E + ebase
            dst0 = (2 * r + 1) * E + ebase

            if k_rem:
                srcr_v, dstr_v, rowsr_v = rem_bufs
            pltpu.sync_copy(ei_hbm.at[pl.ds(src0, n_full * K)], src_v)
            pltpu.sync_copy(ei_hbm.at[pl.ds(dst0, K)], dstb[0])
            for b in range(n_full):
                cur = dstb[b % 2]
                nxt = dstb[(b + 1) % 2]
                if b + 1 < n_full:
                    pltpu.async_copy(
                        ei_hbm.at[pl.ds(dst0 + (b + 1) * K, K)], nxt, sem_i)
                elif k_rem:
                    pltpu.async_copy(
                        ei_hbm.at[pl.ds(dst0 + n_full * K, k_rem)], dstr_v,
                        sem_i)
                pltpu.async_copy(x_hbm.at[src_v.at[pl.ds(b * K, K)]],
                                 rows_v, sem).wait()
                pltpu.sync_copy(rows_v, acc_sh.at[cur], add=True)
                if b + 1 < n_full:
                    pltpu.make_async_copy(
                        ei_hbm.at[pl.ds(dst0, K)], nxt, sem_i).wait()
                elif k_rem:
                    pltpu.make_async_copy(
                        ei_hbm.at[pl.ds(dst0, k_rem)], dstr_v, sem_i).wait()
            if k_rem:
                off = n_full * K
                pltpu.sync_copy(ei_hbm.at[pl.ds(src0 + off, k_rem)], srcr_v)
                pltpu.async_copy(x_hbm.at[srcr_v], rowsr_v, sem).wait()
                pltpu.sync_copy(rowsr_v, acc_sh.at[dstr_v], add=True)
            plsc.subcore_barrier()

            pltpu.sync_copy(
                acc_sh.at[pl.ds(row0, rpt)],
                out_hbm.at[r, pl.ds(row0, rpt)])
            if tail:
                @pl.when(s < tail // 8)
                def _():
                    pltpu.sync_copy(acc_sh.at[pl.ds(trow, 8)],
                                    out_hbm.at[r, pl.ds(trow, 8)])

    return sc_agg


@functools.lru_cache(maxsize=None)
def _make_tc_self(N, D, Dout, bm=1000):
    def _body(x_ref, ws_ref, o_ref):
        o_ref[...] = jnp.dot(x_ref[...], ws_ref[...],
                             preferred_element_type=jnp.float32)

    return pl.pallas_call(
        _body,
        grid=(N // bm,),
        in_specs=[
            pl.BlockSpec((bm, D), lambda i: (i, 0)),
            pl.BlockSpec((D, Dout), lambda i: (0, 0)),
        ],
        out_specs=pl.BlockSpec((bm, Dout), lambda i: (i, 0)),
        out_shape=jax.ShapeDtypeStruct((N, Dout), jnp.float32),
    )


@functools.lru_cache(maxsize=None)
def _make_tc_sum(N, D, Dout, R, bm=1000):
    def _body(s_ref, a_ref, w_ref, o_ref):
        acc = s_ref[...]
        for r in range(R):
            acc = acc + jnp.dot(a_ref[r], w_ref[r],
                                preferred_element_type=jnp.float32)
        o_ref[...] = acc

    return pl.pallas_call(
        _body,
        grid=(N // bm,),
        in_specs=[
            pl.BlockSpec((bm, Dout), lambda i: (i, 0)),
            pl.BlockSpec((R, bm, D), lambda i: (0, i, 0)),
            pl.BlockSpec((R, D, Dout), lambda i: (0, 0, 0)),
        ],
        out_specs=pl.BlockSpec((bm, Dout), lambda i: (i, 0)),
        out_shape=jax.ShapeDtypeStruct((N, Dout), jnp.float32),
    )


def kernel(x, edge_index, W, W_self):
    B, N, D = x.shape
    R, _, E = edge_index.shape
    Dout = W_self.shape[1]
    x2 = x.reshape(N, D)
    self_o = _make_tc_self(N, D, Dout)(x2, W_self)
    agg = _make_sc_agg(N, D, R, E)(x2, edge_index.reshape(-1))
    out = _make_tc_sum(N, D, Dout, R)(self_o, agg, W)
    return out.reshape(B, N, Dout)


# speedup vs baseline: 3.3840x; 1.4302x over previous
import functools

import jax
import jax.numpy as jnp
from jax import lax
from jax.experimental import pallas as pl
from jax.experimental.pallas import tpu as pltpu
from jax.experimental.pallas import tpu_sc as plsc

NC = 2
NS = 16
K = 128


@functools.lru_cache(maxsize=None)
def _make_sc_agg(N, D, R, E):
    assert R % NC == 0
    rel_per_core = R // NC
    e_per_tile = E // NS
    n_full = e_per_tile // K
    k_rem = e_per_tile - n_full * K
    rpt = (N // NS) // 8 * 8
    tail = N - NS * rpt
    assert tail % 8 == 0 and tail // 8 <= NS
    n_z128 = rpt // K
    z_rem = rpt - n_z128 * K

    mesh = plsc.VectorSubcoreMesh(core_axis_name="c", subcore_axis_name="s")

    scratch = [
        pltpu.VMEM((n_full * K,), jnp.int32),
        pltpu.VMEM((K,), jnp.int32),
        pltpu.VMEM((K,), jnp.int32),
        pltpu.VMEM((K, D), jnp.float32),
        pltpu.VMEM((K, D), jnp.float32),
        pltpu.VMEM_SHARED((N, D), jnp.float32),
        pltpu.SemaphoreType.DMA,
        pltpu.SemaphoreType.DMA,
        pltpu.SemaphoreType.DMA,
    ]
    if k_rem:
        scratch += [
            pltpu.VMEM((k_rem,), jnp.int32),
            pltpu.VMEM((k_rem,), jnp.int32),
            pltpu.VMEM((k_rem, D), jnp.float32),
        ]

    @functools.partial(
        pl.kernel,
        mesh=mesh,
        out_type=jax.ShapeDtypeStruct((R, N, D), jnp.float32),
        scratch_types=scratch,
    )
    def sc_agg(x_hbm, ei_hbm, out_hbm, src_v, dst_a, dst_b, rows_a, rows_b,
               acc_sh, sem_a, sem_b, sem_i, *rem_bufs):
        dstb = (dst_a, dst_b)
        rowsb = (rows_a, rows_b)
        sems = (sem_a, sem_b)
        rows_v = rows_a
        c = lax.axis_index("c")
        s = lax.axis_index("s")
        ebase = s * e_per_tile
        row0 = s * rpt
        trow = NS * rpt + s * 8

        for phase in range(rel_per_core):
            r = c * rel_per_core + phase

            def _zrow(i, carry):
                for j in range(D // 16):
                    rows_v[i, pl.ds(j * 16, 16)] = jnp.zeros((16,), jnp.float32)
                return carry
            lax.fori_loop(0, K, _zrow, 0)
            for z in range(n_z128):
                pltpu.sync_copy(
                    rows_v,
                    acc_sh.at[pl.ds(row0 + z * K, K)])
            if z_rem:
                pltpu.sync_copy(
                    rows_v.at[pl.ds(0, z_rem)],
                    acc_sh.at[pl.ds(row0 + n_z128 * K, z_rem)])
            if tail:
                @pl.when(s < tail // 8)
                def _():
                    pltpu.sync_copy(rows_v.at[pl.ds(0, 8)],
                                    acc_sh.at[pl.ds(trow, 8)])
            plsc.subcore_barrier()

            src0 = (2 * r) * E + ebase
            dst0 = (2 * r + 1) * E + ebase

            if k_rem:
                srcr_v, dstr_v, rowsr_v = rem_bufs
            pltpu.sync_copy(ei_hbm.at[pl.ds(src0, n_full * K)], src_v)
            pltpu.sync_copy(ei_hbm.at[pl.ds(dst0, K)], dstb[0])
            if k_rem:
                pltpu.sync_copy(ei_hbm.at[pl.ds(src0 + n_full * K, k_rem)],
                                srcr_v)
            pltpu.async_copy(x_hbm.at[src_v.at[pl.ds(0, K)]], rowsb[0],
                             sems[0])
            for b in range(n_full):
                cur = dstb[b % 2]
                nxt = dstb[(b + 1) % 2]
                rb = rowsb[b % 2]
                rn = rowsb[(b + 1) % 2]
                if b + 1 < n_full:
                    pltpu.async_copy(
                        ei_hbm.at[pl.ds(dst0 + (b + 1) * K, K)], nxt, sem_i)
                    pltpu.async_copy(
                        x_hbm.at[src_v.at[pl.ds((b + 1) * K, K)]], rn,
                        sems[(b + 1) % 2])
                elif k_rem:
                    pltpu.async_copy(
                        ei_hbm.at[pl.ds(dst0 + n_full * K, k_rem)], dstr_v,
                        sem_i)
                    pltpu.async_copy(x_hbm.at[srcr_v], rowsr_v,
                                     sems[(b + 1) % 2])
                pltpu.make_async_copy(x_hbm.at[src_v.at[pl.ds(b * K, K)]],
                                      rb, sems[b % 2]).wait()
                pltpu.sync_copy(rb, acc_sh.at[cur], add=True)
                if b + 1 < n_full:
                    pltpu.make_async_copy(
                        ei_hbm.at[pl.ds(dst0, K)], nxt, sem_i).wait()
                elif k_rem:
                    pltpu.make_async_copy(
                        ei_hbm.at[pl.ds(dst0, k_rem)], dstr_v, sem_i).wait()
            if k_rem:
                pltpu.make_async_copy(x_hbm.at[srcr_v], rowsr_v,
                                      sems[n_full % 2]).wait()
                pltpu.sync_copy(rowsr_v, acc_sh.at[dstr_v], add=True)
            plsc.subcore_barrier()

            pltpu.sync_copy(
                acc_sh.at[pl.ds(row0, rpt)],
                out_hbm.at[r, pl.ds(row0, rpt)])
            if tail:
                @pl.when(s < tail // 8)
                def _():
                    pltpu.sync_copy(acc_sh.at[pl.ds(trow, 8)],
                                    out_hbm.at[r, pl.ds(trow, 8)])

    return sc_agg


@functools.lru_cache(maxsize=None)
def _make_tc_self(N, D, Dout, bm=1000):
    def _body(x_ref, ws_ref, o_ref):
        o_ref[...] = jnp.dot(x_ref[...], ws_ref[...],
                             preferred_element_type=jnp.float32)

    return pl.pallas_call(
        _body,
        grid=(N // bm,),
        in_specs=[
            pl.BlockSpec((bm, D), lambda i: (i, 0)),
            pl.BlockSpec((D, Dout), lambda i: (0, 0)),
        ],
        out_specs=pl.BlockSpec((bm, Dout), lambda i: (i, 0)),
        out_shape=jax.ShapeDtypeStruct((N, Dout), jnp.float32),
    )


@functools.lru_cache(maxsize=None)
def _make_tc_sum(N, D, Dout, R, bm=1000):
    def _body(s_ref, a_ref, w_ref, o_ref):
        acc = s_ref[...]
        for r in range(R):
            acc = acc + jnp.dot(a_ref[r], w_ref[r],
                                preferred_element_type=jnp.float32)
        o_ref[...] = acc

    return pl.pallas_call(
        _body,
        grid=(N // bm,),
        in_specs=[
            pl.BlockSpec((bm, Dout), lambda i: (i, 0)),
            pl.BlockSpec((R, bm, D), lambda i: (0, i, 0)),
            pl.BlockSpec((R, D, Dout), lambda i: (0, 0, 0)),
        ],
        out_specs=pl.BlockSpec((bm, Dout), lambda i: (i, 0)),
        out_shape=jax.ShapeDtypeStruct((N, Dout), jnp.float32),
    )


def kernel(x, edge_index, W, W_self):
    B, N, D = x.shape
    R, _, E = edge_index.shape
    Dout = W_self.shape[1]
    x2 = x.reshape(N, D)
    self_o = _make_tc_self(N, D, Dout)(x2, W_self)
    agg = _make_sc_agg(N, D, R, E)(x2, edge_index.reshape(-1))
    out = _make_tc_sum(N, D, Dout, R)(self_o, agg, W)
    return out.reshape(B, N, Dout)
